# baseline (speedup 1.0000x reference)
import functools

import jax
import jax.numpy as jnp
from jax import lax
from jax.experimental import pallas as pl
from jax.experimental.pallas import tpu as pltpu
from jax.experimental.pallas import tpu_sc as plsc

N = 10000
F = 128
NFEAT = 256
E = 160000
NC, NS, L = 2, 16, 16
NW = NC * NS
C = 128
NCHUNK = E // C
NPAD = 10240
ROWS_PER_SUB = NPAD // NS
ZROWS = 128


def _linear_body(x_ref, w_ref, b_ref, o_ref):
    o_ref[...] = lax.dot_general(
        x_ref[...], w_ref[...], (((1,), (1,)), ((), ())),
        preferred_element_type=jnp.float32) + b_ref[...]


def _linear(x, W, b2):
    blk = 1000
    return pl.pallas_call(
        _linear_body,
        grid=(N // blk,),
        in_specs=[pl.BlockSpec((blk, NFEAT), lambda i: (i, 0)),
                  pl.BlockSpec((F, NFEAT), lambda i: (0, 0)),
                  pl.BlockSpec((1, F), lambda i: (0, 0))],
        out_specs=pl.BlockSpec((blk, F), lambda i: (i, 0)),
        out_shape=jax.ShapeDtypeStruct((N, F), jnp.float32),
    )(x, W, b2)


_GATHER_DN = lax.GatherDimensionNumbers(
    offset_dims=(), collapsed_slice_dims=(0,), start_index_map=(0,))


def _bcast_lane(vec, e):
    idx = jnp.full((L, 1), e, jnp.int32)
    return lax.gather(vec, idx, _GATHER_DN, (1,),
                      mode=lax.GatherScatterMode.PROMISE_IN_BOUNDS)


def _spmm_body(h_hbm, e_hbm, w_hbm, z_hbm, out_hbm, ebuf, w_v, rows_v, acc_sh, sem):
    cid = lax.axis_index("c")
    sid = lax.axis_index("s")
    wid = sid * NC + cid

    pltpu.sync_copy(z_hbm,
                    acc_sh.at[pl.ds(sid * ROWS_PER_SUB, ROWS_PER_SUB)])
    plsc.subcore_barrier()

    nchunks = 39 + jnp.where(wid < NCHUNK - 39 * NW, 1, 0)

    def chunk(k, _):
        crow = wid + NW * k
        pltpu.sync_copy(e_hbm.at[crow], ebuf)
        pltpu.sync_copy(w_hbm.at[pl.ds(crow * C, C)], w_v)
        pltpu.async_copy(h_hbm.at[ebuf.at[0]], rows_v, sem).wait()

        def scale(g, _):
            wv = w_v[pl.ds(g * L, L)]
            for e in range(L):
                ws = _bcast_lane(wv, e)
                r = g * L + e
                for j in range(F // L):
                    sl = pl.ds(j * L, L)
                    rows_v[r, sl] = rows_v[r, sl] * ws
            return 0
        lax.fori_loop(0, C // L, scale, 0)

        pltpu.sync_copy(rows_v, acc_sh.at[ebuf.at[1]], add=True)
        return 0
    lax.fori_loop(0, nchunks, chunk, 0)
    plsc.subcore_barrier()

    rbase = sid * ROWS_PER_SUB
    pltpu.sync_copy(acc_sh.at[pl.ds(rbase, ROWS_PER_SUB)],
                    out_hbm.at[cid, pl.ds(rbase, ROWS_PER_SUB)])


_spmm = functools.partial(
    pl.kernel,
    out_type=jax.ShapeDtypeStruct((NC, NPAD, F), jnp.float32),
    mesh=plsc.VectorSubcoreMesh(core_axis_name="c", subcore_axis_name="s",
                                num_cores=NC, num_subcores=NS),
    scratch_types=[
        pltpu.VMEM((2, C), jnp.int32),
        pltpu.VMEM((C,), jnp.float32),
        pltpu.VMEM((C, F), jnp.float32),
        pltpu.VMEM_SHARED((NPAD, F), jnp.float32),
        pltpu.SemaphoreType.DMA,
    ],
)(_spmm_body)


def _add_body(a_ref, b_ref, o_ref):
    o_ref[...] = a_ref[0] + b_ref[0]


def _combine(p):
    blk = 1000
    return pl.pallas_call(
        _add_body,
        grid=(N // blk,),
        in_specs=[pl.BlockSpec((1, blk, F), lambda i: (0, i, 0)),
                  pl.BlockSpec((1, blk, F), lambda i: (1, i, 0))],
        out_specs=pl.BlockSpec((blk, F), lambda i: (i, 0)),
        out_shape=jax.ShapeDtypeStruct((N, F), jnp.float32),
    )(p, p)


def _lsm_body(a_ref, b_ref, o_ref):
    h = a_ref[0] + b_ref[0]
    m = jnp.max(h, axis=1, keepdims=True)
    ex = jnp.exp(h - m)
    s = jnp.sum(ex, axis=1, keepdims=True)
    o_ref[...] = h - m - jnp.log(s)


def _combine_lsm(p):
    blk = 1000
    return pl.pallas_call(
        _lsm_body,
        grid=(N // blk,),
        in_specs=[pl.BlockSpec((1, blk, F), lambda i: (0, i, 0)),
                  pl.BlockSpec((1, blk, F), lambda i: (1, i, 0))],
        out_specs=pl.BlockSpec((blk, F), lambda i: (i, 0)),
        out_shape=jax.ShapeDtypeStruct((N, F), jnp.float32),
    )(p, p)


def kernel(x, edge_index, edge_weight, W, b):
    src = edge_index[1].astype(jnp.int32).reshape(NCHUNK, 1, C)
    dst = edge_index[0].astype(jnp.int32).reshape(NCHUNK, 1, C)
    epack = jnp.concatenate([src, dst], axis=1)
    w = edge_weight.astype(jnp.float32)
    z = jnp.zeros((ROWS_PER_SUB, F), jnp.float32)
    h = _linear(x, W, b.reshape(1, F).astype(jnp.float32))
    p = _spmm(h, epack, w, z)
    h = _combine(p)
    p = _spmm(h, epack, w, z)
    return _combine_lsm(p)


# speedup vs baseline: 1.0221x; 1.0221x over previous
import functools

import jax
import jax.numpy as jnp
from jax import lax
from jax.experimental import pallas as pl
from jax.experimental.pallas import tpu as pltpu
from jax.experimental.pallas import tpu_sc as plsc

N = 10000
F = 128
NFEAT = 256
E = 160000
NC, NS, L = 2, 16, 16
NW = NC * NS
C = 128
NCHUNK = E // C
NPAD = 10240
ROWS_PER_SUB = NPAD // NS
ZROWS = 128


def _linear_body(x_ref, w_ref, b_ref, o_ref):
    o_ref[...] = lax.dot_general(
        x_ref[...], w_ref[...], (((1,), (1,)), ((), ())),
        preferred_element_type=jnp.float32) + b_ref[...]


def _linear(x, W, b2):
    blk = 1000
    return pl.pallas_call(
        _linear_body,
        grid=(N // blk,),
        in_specs=[pl.BlockSpec((blk, NFEAT), lambda i: (i, 0)),
                  pl.BlockSpec((F, NFEAT), lambda i: (0, 0)),
                  pl.BlockSpec((1, F), lambda i: (0, 0))],
        out_specs=pl.BlockSpec((blk, F), lambda i: (i, 0)),
        out_shape=jax.ShapeDtypeStruct((N, F), jnp.float32),
    )(x, W, b2)


_GATHER_DN = lax.GatherDimensionNumbers(
    offset_dims=(), collapsed_slice_dims=(0,), start_index_map=(0,))


def _bcast_lane(vec, e):
    idx = jnp.full((L, 1), e, jnp.int32)
    return lax.gather(vec, idx, _GATHER_DN, (1,),
                      mode=lax.GatherScatterMode.PROMISE_IN_BOUNDS)


def _spmm_body(h_hbm, e_hbm, w_hbm, out_hbm, ebuf, w_v, rows_v, acc_sh, sem):
    cid = lax.axis_index("c")
    sid = lax.axis_index("s")
    wid = sid * NC + cid

    def zrow(i, _):
        def zcol(j, _):
            rows_v[i, pl.ds(j * L, L)] = jnp.zeros((L,), jnp.float32)
            return 0
        return lax.fori_loop(0, F // L, zcol, 0)
    lax.fori_loop(0, ZROWS, zrow, 0)

    def zblk(t, _):
        pltpu.sync_copy(rows_v,
                        acc_sh.at[pl.ds(sid * ROWS_PER_SUB + t * ZROWS, ZROWS)])
        return 0
    lax.fori_loop(0, ROWS_PER_SUB // ZROWS, zblk, 0)
    plsc.subcore_barrier()

    nchunks = 39 + jnp.where(wid < NCHUNK - 39 * NW, 1, 0)

    def chunk(k, _):
        crow = wid + NW * k
        pltpu.sync_copy(e_hbm.at[crow], ebuf)
        pltpu.sync_copy(w_hbm.at[pl.ds(crow * C, C)], w_v)
        pltpu.async_copy(h_hbm.at[ebuf.at[0]], rows_v, sem).wait()

        def scale(g, _):
            wv = w_v[pl.ds(g * L, L)]
            for e in range(L):
                ws = _bcast_lane(wv, e)
                r = g * L + e
                for j in range(F // L):
                    sl = pl.ds(j * L, L)
                    rows_v[r, sl] = rows_v[r, sl] * ws
            return 0
        lax.fori_loop(0, C // L, scale, 0)

        pltpu.sync_copy(rows_v, acc_sh.at[ebuf.at[1]], add=True)
        return 0
    lax.fori_loop(0, nchunks, chunk, 0)
    plsc.subcore_barrier()

    rbase = sid * ROWS_PER_SUB
    pltpu.sync_copy(acc_sh.at[pl.ds(rbase, ROWS_PER_SUB)],
                    out_hbm.at[cid, pl.ds(rbase, ROWS_PER_SUB)])


_spmm = functools.partial(
    pl.kernel,
    out_type=jax.ShapeDtypeStruct((NC, NPAD, F), jnp.float32),
    mesh=plsc.VectorSubcoreMesh(core_axis_name="c", subcore_axis_name="s",
                                num_cores=NC, num_subcores=NS),
    scratch_types=[
        pltpu.VMEM((2, C), jnp.int32),
        pltpu.VMEM((C,), jnp.float32),
        pltpu.VMEM((C, F), jnp.float32),
        pltpu.VMEM_SHARED((NPAD, F), jnp.float32),
        pltpu.SemaphoreType.DMA,
    ],
)(_spmm_body)


def _add_body(a_ref, b_ref, o_ref):
    o_ref[...] = a_ref[0] + b_ref[0]


def _combine(p):
    blk = 1000
    return pl.pallas_call(
        _add_body,
        grid=(N // blk,),
        in_specs=[pl.BlockSpec((1, blk, F), lambda i: (0, i, 0)),
                  pl.BlockSpec((1, blk, F), lambda i: (1, i, 0))],
        out_specs=pl.BlockSpec((blk, F), lambda i: (i, 0)),
        out_shape=jax.ShapeDtypeStruct((N, F), jnp.float32),
    )(p, p)


def _lsm_body(a_ref, b_ref, o_ref):
    h = a_ref[0] + b_ref[0]
    m = jnp.max(h, axis=1, keepdims=True)
    ex = jnp.exp(h - m)
    s = jnp.sum(ex, axis=1, keepdims=True)
    o_ref[...] = h - m - jnp.log(s)


def _combine_lsm(p):
    blk = 1000
    return pl.pallas_call(
        _lsm_body,
        grid=(N // blk,),
        in_specs=[pl.BlockSpec((1, blk, F), lambda i: (0, i, 0)),
                  pl.BlockSpec((1, blk, F), lambda i: (1, i, 0))],
        out_specs=pl.BlockSpec((blk, F), lambda i: (i, 0)),
        out_shape=jax.ShapeDtypeStruct((N, F), jnp.float32),
    )(p, p)


def kernel(x, edge_index, edge_weight, W, b):
    src = edge_index[1].astype(jnp.int32).reshape(NCHUNK, 1, C)
    dst = edge_index[0].astype(jnp.int32).reshape(NCHUNK, 1, C)
    epack = jnp.concatenate([src, dst], axis=1)
    w = edge_weight.astype(jnp.float32)
    h = _linear(x, W, b.reshape(1, F).astype(jnp.float32))
    p = _spmm(h, epack, w)
    h = _combine(p)
    p = _spmm(h, epack, w)
    return _combine_lsm(p)
